# baseline (speedup 1.0000x reference)
import functools

import jax
import jax.numpy as jnp
from jax import lax
from jax.experimental import pallas as pl
from jax.experimental.pallas import tpu as pltpu
from jax.experimental.pallas import tpu_sc as plsc

_B = 4096
_S = 50
_D = 64
_NC = 2
_NS = 16
_NW = _NC * _NS
_BW = _B // _NW
_NBUF = 6
_NITER = -(-_S // _NBUF)
_KB = _BW // 16

_mesh = plsc.VectorSubcoreMesh(core_axis_name="c", subcore_axis_name="s")


@functools.partial(
    pl.kernel,
    mesh=_mesh,
    compiler_params=pltpu.CompilerParams(needs_layout_passes=False),
    out_type=jax.ShapeDtypeStruct((_S, _D, _B), jnp.float32),
    scratch_types=[
        pltpu.VMEM((_S, _BW), jnp.int32),
        pltpu.VMEM((_S, _BW), jnp.int32),
        [pltpu.VMEM((_BW, 2 * _D), jnp.float32) for _ in range(_NBUF)],
        [pltpu.VMEM((_D, _BW), jnp.float32) for _ in range(2)],
        [pltpu.SemaphoreType.DMA for _ in range(_NBUF)],
        [pltpu.SemaphoreType.DMA for _ in range(2)],
    ],
)
def _gather_kernel(xT_hbm, tab_hbm, out_hbm, idx_v, par_v, gbufs, tbufs,
                   gsems, ssems):
    w = lax.axis_index("s") * _NC + lax.axis_index("c")
    b0 = w * _BW
    pltpu.sync_copy(xT_hbm.at[:, pl.ds(b0, _BW)], idx_v)

    def prep(s, carry):
        for kb in range(_KB):
            v = idx_v[s, pl.ds(kb * 16, 16)]
            par_v[s, pl.ds(kb * 16, 16)] = (v & 1) * 64
            idx_v[s, pl.ds(kb * 16, 16)] = lax.shift_right_logical(v, 1)
        return carry

    lax.fori_loop(0, _S, prep, 0)

    for b in range(_NBUF):
        pltpu.async_copy(tab_hbm.at[idx_v.at[b]], gbufs[b], gsems[b])

    iota16 = lax.broadcasted_iota(jnp.int32, (16,), 0)
    rows16 = [iota16 + kb * 16 for kb in range(_KB)]

    def out_slice(s):
        return out_hbm.at[s, :, pl.ds(b0, _BW)]

    def body(i, carry):
        s0 = i * _NBUF
        for b in range(_NBUF):
            s = s0 + b
            tr = b % 2

            @pl.when(s < _S)
            def _():
                pltpu.make_async_copy(
                    tab_hbm.at[idx_v.at[s]], gbufs[b], gsems[b]
                ).wait()

                @pl.when(s >= 2)
                def _():
                    pltpu.make_async_copy(
                        tbufs[tr], out_slice(s - 2), ssems[tr]
                    ).wait()

                base = [
                    par_v[s, pl.ds(kb * 16, 16)] for kb in range(_KB)
                ]

                def transpose_diag(d, carry2):
                    perm = (iota16 + d) & 15
                    for c0 in range(0, _D, 16):
                        crow = perm + c0
                        for kb in range(_KB):
                            vals = plsc.load_gather(
                                gbufs[b], [rows16[kb], base[kb] + crow]
                            )
                            plsc.store_scatter(
                                tbufs[tr], [crow, rows16[kb]], vals
                            )
                    return carry2

                lax.fori_loop(0, 16, transpose_diag, 0)
                pltpu.async_copy(tbufs[tr], out_slice(s), ssems[tr])

                @pl.when(s + _NBUF < _S)
                def _():
                    pltpu.async_copy(
                        tab_hbm.at[idx_v.at[s + _NBUF]], gbufs[b], gsems[b]
                    )

        return carry

    lax.fori_loop(0, _NITER, body, 0)

    for tr in range(2):
        pltpu.make_async_copy(
            tbufs[tr], out_slice(_S - 2 + tr), ssems[tr]
        ).wait()


def kernel(x, table):
    xT = x.astype(jnp.int32).T
    tab = table.reshape(table.shape[0] // 2, 2 * _D)
    o = _gather_kernel(xT, tab)
    return o.transpose(2, 0, 1)


# speedup vs baseline: 1.0020x; 1.0020x over previous
import functools

import jax
import jax.numpy as jnp
from jax import lax
from jax.experimental import pallas as pl
from jax.experimental.pallas import tpu as pltpu
from jax.experimental.pallas import tpu_sc as plsc

_B = 4096
_S = 50
_D = 64
_NC = 2
_NS = 16
_NW = _NC * _NS
_BW = _B // _NW
_NBUF = 4
_NITER = -(-_S // _NBUF)
_KB = _BW // 16

_mesh = plsc.VectorSubcoreMesh(core_axis_name="c", subcore_axis_name="s")


@functools.partial(
    pl.kernel,
    mesh=_mesh,
    compiler_params=pltpu.CompilerParams(needs_layout_passes=False),
    out_type=jax.ShapeDtypeStruct((_S, _D, _B), jnp.float32),
    scratch_types=[
        pltpu.VMEM((_S, _BW), jnp.int32),
        pltpu.VMEM((_S, _BW), jnp.int32),
        [pltpu.VMEM((_BW, 2 * _D), jnp.float32) for _ in range(_NBUF)],
        [pltpu.VMEM((_D, _BW), jnp.float32) for _ in range(2)],
        [pltpu.SemaphoreType.DMA for _ in range(_NBUF)],
        [pltpu.SemaphoreType.DMA for _ in range(2)],
    ],
)
def _gather_kernel(xT_hbm, tab_hbm, out_hbm, idx_v, par_v, gbufs, tbufs,
                   gsems, ssems):
    w = lax.axis_index("s") * _NC + lax.axis_index("c")
    b0 = w * _BW
    pltpu.sync_copy(xT_hbm.at[:, pl.ds(b0, _BW)], idx_v)

    def prep(s, carry):
        for kb in range(_KB):
            v = idx_v[s, pl.ds(kb * 16, 16)]
            par_v[s, pl.ds(kb * 16, 16)] = (v & 1) * 64
            idx_v[s, pl.ds(kb * 16, 16)] = lax.shift_right_logical(v, 1)
        return carry

    lax.fori_loop(0, _S, prep, 0)

    for b in range(_NBUF):
        pltpu.async_copy(tab_hbm.at[idx_v.at[b]], gbufs[b], gsems[b])

    iota16 = lax.broadcasted_iota(jnp.int32, (16,), 0)
    rows16 = [iota16 + kb * 16 for kb in range(_KB)]

    def out_slice(s):
        return out_hbm.at[s, :, pl.ds(b0, _BW)]

    def body(i, carry):
        s0 = i * _NBUF
        for b in range(_NBUF):
            s = s0 + b
            tr = b % 2

            @pl.when(s < _S)
            def _():
                pltpu.make_async_copy(
                    tab_hbm.at[idx_v.at[s]], gbufs[b], gsems[b]
                ).wait()

                @pl.when(s >= 2)
                def _():
                    pltpu.make_async_copy(
                        tbufs[tr], out_slice(s - 2), ssems[tr]
                    ).wait()

                base = [
                    par_v[s, pl.ds(kb * 16, 16)] for kb in range(_KB)
                ]

                def transpose_diag(d, carry2):
                    perm = (iota16 + d) & 15
                    for kb in range(_KB):
                        for c0 in range(0, _D, 16):
                            crow = perm + c0
                            vals = plsc.load_gather(
                                gbufs[b], [rows16[kb], base[kb] + crow]
                            )
                            plsc.store_scatter(
                                tbufs[tr], [crow, rows16[kb]], vals
                            )
                    return carry2

                lax.fori_loop(0, 16, transpose_diag, 0)
                pltpu.async_copy(tbufs[tr], out_slice(s), ssems[tr])

                @pl.when(s + _NBUF < _S)
                def _():
                    pltpu.async_copy(
                        tab_hbm.at[idx_v.at[s + _NBUF]], gbufs[b], gsems[b]
                    )

        return carry

    lax.fori_loop(0, _NITER, body, 0)

    for tr in range(2):
        pltpu.make_async_copy(
            tbufs[tr], out_slice(_S - 2 + tr), ssems[tr]
        ).wait()


def kernel(x, table):
    xT = x.astype(jnp.int32).T
    tab = table.reshape(table.shape[0] // 2, 2 * _D)
    o = _gather_kernel(xT, tab)
    return o.transpose(2, 0, 1)
